# baseline (speedup 1.0000x reference)
import functools

import jax
import jax.numpy as jnp
import numpy as np
from jax import lax
from jax.experimental import pallas as pl
from jax.experimental.pallas import tpu as pltpu
from jax.experimental.pallas import tpu_sc as plsc

_MAX_DEGREE = 10000.0
_K = 128
_L = 16
_NC = 2
_NS = 16
_NW = _NC * _NS

_QV = np.linspace(0.0, _MAX_DEGREE, _K).astype(np.float32)
_CS = np.cumsum(_QV, dtype=np.float32)
_RS = np.concatenate([_QV[1:], _QV[:1]]).astype(np.float32)


@functools.cache
def _build(b, s):
    n = b * s
    npw = n // _NW
    rows = b // _NW
    nv = npw // _L

    mesh = plsc.VectorSubcoreMesh(core_axis_name="c", subcore_axis_name="s")

    @functools.partial(
        pl.kernel,
        mesh=mesh,
        out_type=jax.ShapeDtypeStruct((b, 2 * s), jnp.float32),
        scratch_types=[
            pltpu.VMEM((npw,), jnp.float32),
            pltpu.VMEM((2 * npw,), jnp.float32),
            pltpu.VMEM((_K,), jnp.float32),
            pltpu.VMEM((_K,), jnp.float32),
        ],
        compiler_params=pltpu.CompilerParams(needs_layout_passes=False),
    )
    def run(deg_hbm, cs_hbm, rs_hbm, out_hbm, d_buf, o_buf, cs_buf, rs_buf):
        wid = lax.axis_index("s") * _NC + lax.axis_index("c")
        base = wid * npw
        pltpu.sync_copy(cs_hbm, cs_buf)
        pltpu.sync_copy(rs_hbm, rs_buf)
        pltpu.sync_copy(deg_hbm.at[pl.ds(base, npw)], d_buf)

        lane2 = lax.iota(jnp.int32, _L) * 2

        @plsc.parallel_loop(0, nv, step=1, unroll=16)
        def body(j):
            d = d_buf[pl.ds(j * _L, _L)]
            k = jnp.zeros((_L,), jnp.int32)
            for st in (64, 32, 16, 8, 4, 2, 1):
                cand = k + st
                v = plsc.load_gather(cs_buf, [cand])
                k = jnp.where(d >= v, cand, k)
            start = plsc.load_gather(cs_buf, [k])
            resid = plsc.load_gather(rs_buf, [k])
            idx = jnp.where(d >= 0.0, k, -1)
            rate = (d - start) / (resid + 1e-10)
            rate = jnp.minimum(jnp.maximum(rate, 0.0), 1.0)
            off = lane2 + j * (2 * _L)
            plsc.store_scatter(o_buf, [off], idx.astype(jnp.float32))
            plsc.store_scatter(o_buf, [off + 1], rate)

        for r in range(rows):
            pltpu.sync_copy(o_buf.at[pl.ds(r * 2 * s, 2 * s)],
                            out_hbm.at[wid * rows + r])

    return run


def kernel(degrees):
    b, s, _ = degrees.shape
    flat = degrees.reshape(b * s)
    out = _build(b, s)(flat, jnp.asarray(_CS), jnp.asarray(_RS))
    return out.reshape(b, s, 2)


# speedup vs baseline: 1.4041x; 1.4041x over previous
import functools

import jax
import jax.numpy as jnp
import numpy as np
from jax import lax
from jax.experimental import pallas as pl
from jax.experimental.pallas import tpu as pltpu
from jax.experimental.pallas import tpu_sc as plsc

_MAX_DEGREE = 10000.0
_K = 128
_L = 16
_NC = 2
_NS = 16
_NW = _NC * _NS

_QV = np.linspace(0.0, _MAX_DEGREE, _K).astype(np.float32)
_CS = np.cumsum(_QV, dtype=np.float32)
_RS = np.concatenate([_QV[1:], _QV[:1]]).astype(np.float32)


@functools.cache
def _build(b, s):
    n = b * s
    npw = n // _NW
    rows = b // _NW
    vpr = s // _L
    nv = npw // _L

    mesh = plsc.VectorSubcoreMesh(core_axis_name="c", subcore_axis_name="s")

    @functools.partial(
        pl.kernel,
        mesh=mesh,
        out_type=jax.ShapeDtypeStruct((b, 2, s), jnp.float32),
        scratch_types=[
            pltpu.VMEM((npw,), jnp.float32),
            pltpu.VMEM((2 * npw,), jnp.float32),
            pltpu.VMEM((_K,), jnp.float32),
            pltpu.VMEM((_K,), jnp.float32),
        ],
        compiler_params=pltpu.CompilerParams(needs_layout_passes=False),
    )
    def run(deg_hbm, cs_hbm, rs_hbm, out_hbm, d_buf, o_buf, cs_buf, rs_buf):
        wid = lax.axis_index("s") * _NC + lax.axis_index("c")
        base = wid * npw
        pltpu.sync_copy(cs_hbm, cs_buf)
        pltpu.sync_copy(rs_hbm, rs_buf)
        pltpu.sync_copy(deg_hbm.at[pl.ds(base, npw)], d_buf)

        @plsc.parallel_loop(0, nv, step=1, unroll=8)
        def body(j):
            d = d_buf[pl.ds(j * _L, _L)]
            k = jnp.zeros((_L,), jnp.int32)
            for st in (64, 32, 16, 8, 4, 2, 1):
                cand = k + st
                v = plsc.load_gather(cs_buf, [cand])
                k = jnp.where(d >= v, cand, k)
            start = plsc.load_gather(cs_buf, [k])
            resid = plsc.load_gather(rs_buf, [k])
            idx = jnp.where(d >= 0.0, k, -1)
            rate = (d - start) / (resid + 1e-10)
            rate = jnp.minimum(jnp.maximum(rate, 0.0), 1.0)
            off = (j // vpr) * (2 * s) + (j % vpr) * _L
            o_buf[pl.ds(off, _L)] = idx.astype(jnp.float32)
            o_buf[pl.ds(off + s, _L)] = rate

        for r in range(rows):
            row = wid * rows + r
            pltpu.sync_copy(o_buf.at[pl.ds(r * 2 * s, s)], out_hbm.at[row, 0])
            pltpu.sync_copy(o_buf.at[pl.ds(r * 2 * s + s, s)], out_hbm.at[row, 1])

    return run


def kernel(degrees):
    b, s, _ = degrees.shape
    flat = degrees.reshape(b * s)
    out = _build(b, s)(flat, jnp.asarray(_CS), jnp.asarray(_RS))
    return jnp.swapaxes(out, 1, 2)


# speedup vs baseline: 1.4818x; 1.0553x over previous
import functools

import jax
import jax.numpy as jnp
import numpy as np
from jax import lax
from jax.experimental import pallas as pl
from jax.experimental.pallas import tpu as pltpu
from jax.experimental.pallas import tpu_sc as plsc

_MAX_DEGREE = 10000.0
_K = 128
_L = 16
_NC = 2
_NS = 16
_NW = _NC * _NS

_QV = np.linspace(0.0, _MAX_DEGREE, _K).astype(np.float32)
_CS = np.cumsum(_QV, dtype=np.float32)
_RS = np.concatenate([_QV[1:], _QV[:1]]).astype(np.float32)
_RINV = (1.0 / (_RS + np.float32(1e-10)).astype(np.float32)).astype(np.float32)


@functools.cache
def _build(b, s):
    n = b * s
    npw = n // _NW
    rows = b // _NW
    vpr = s // _L
    nv = npw // _L

    mesh = plsc.VectorSubcoreMesh(core_axis_name="c", subcore_axis_name="s")

    @functools.partial(
        pl.kernel,
        mesh=mesh,
        out_type=jax.ShapeDtypeStruct((b, 2, s), jnp.float32),
        scratch_types=[
            pltpu.VMEM((npw,), jnp.float32),
            pltpu.VMEM((2 * npw,), jnp.float32),
            pltpu.VMEM((_K,), jnp.float32),
            pltpu.VMEM((_K,), jnp.float32),
            pltpu.SemaphoreType.DMA,
            pltpu.SemaphoreType.DMA,
            pltpu.SemaphoreType.DMA,
        ],
        compiler_params=pltpu.CompilerParams(needs_layout_passes=False),
    )
    def run(deg_hbm, cs_hbm, ri_hbm, out_hbm, d_buf, o_buf, cs_buf, ri_buf,
            sem_a, sem_b, sem_c):
        wid = lax.axis_index("s") * _NC + lax.axis_index("c")
        base = wid * npw
        c1 = pltpu.async_copy(cs_hbm, cs_buf, sem_a)
        c2 = pltpu.async_copy(ri_hbm, ri_buf, sem_b)
        c3 = pltpu.async_copy(deg_hbm.at[pl.ds(base, npw)], d_buf, sem_c)
        c1.wait()
        c2.wait()
        c3.wait()

        @plsc.parallel_loop(0, nv, step=1, unroll=8)
        def body(j):
            d = d_buf[pl.ds(j * _L, _L)]
            k = jnp.zeros((_L,), jnp.int32)
            for st in (64, 32, 16, 8, 4, 2, 1):
                cand = k + st
                v = plsc.load_gather(cs_buf, [cand])
                k = jnp.where(d >= v, cand, k)
            start = plsc.load_gather(cs_buf, [k])
            rinv = plsc.load_gather(ri_buf, [k])
            idx = jnp.where(d >= 0.0, k, -1)
            rate = (d - start) * rinv
            rate = jnp.minimum(jnp.maximum(rate, 0.0), 1.0)
            off = (j // vpr) * (2 * s) + (j % vpr) * _L
            o_buf[pl.ds(off, _L)] = idx.astype(jnp.float32)
            o_buf[pl.ds(off + s, _L)] = rate

        for r in range(rows):
            row = wid * rows + r
            pltpu.sync_copy(o_buf.at[pl.ds(r * 2 * s, s)], out_hbm.at[row, 0])
            pltpu.sync_copy(o_buf.at[pl.ds(r * 2 * s + s, s)], out_hbm.at[row, 1])

    return run


def kernel(degrees):
    b, s, _ = degrees.shape
    flat = degrees.reshape(b * s)
    out = _build(b, s)(flat, jnp.asarray(_CS), jnp.asarray(_RINV))
    return jnp.swapaxes(out, 1, 2)
